# baseline (speedup 1.0000x reference)
import functools

import jax
import jax.numpy as jnp
from jax import lax
from jax.experimental import pallas as pl
from jax.experimental.pallas import tpu as pltpu
from jax.experimental.pallas import tpu_sc as plsc

EMBED_DIM = 16
BATCH = 16384
OUT_DIM = 3

NC = 2
NS = 16
L = 16
NW = NC * NS
BPW = BATCH // NW
IDX_CHUNK = 128
N_CHUNKS = BPW // IDX_CHUNK


@functools.lru_cache(maxsize=1)
def _build_sc_call():
    mesh = plsc.VectorSubcoreMesh(core_axis_name="c", subcore_axis_name="s")

    @functools.partial(
        pl.kernel,
        out_type=jax.ShapeDtypeStruct((BATCH, OUT_DIM), jnp.float32),
        mesh=mesh,
        compiler_params=pltpu.CompilerParams(
            needs_layout_passes=False, use_tc_tiling_on_sc=False),
        scratch_types=[
            pltpu.VMEM((BPW,), jnp.int32),
            pltpu.VMEM((BPW,), jnp.int32),
            pltpu.VMEM((BPW, EMBED_DIM), jnp.float32),
            pltpu.VMEM((BPW, EMBED_DIM), jnp.float32),
            pltpu.VMEM((BPW,), jnp.float32),
            pltpu.VMEM((112,), jnp.float32),
            pltpu.VMEM((BPW, OUT_DIM), jnp.float32),
            pltpu.SemaphoreType.DMA,
        ],
    )
    def sc_call(oid_hbm, pid_hbm, rat_hbm, ptab_hbm, otab_hbm, wb_hbm,
                out_hbm, oid_v, pid_v, orow_v, prow_v, rat_v, wb_v, out_v,
                sem):
        wid = lax.axis_index("c") * NS + lax.axis_index("s")
        base = wid * BPW

        pltpu.sync_copy(oid_hbm.at[pl.ds(base, BPW)], oid_v)
        pltpu.sync_copy(pid_hbm.at[pl.ds(base, BPW)], pid_v)
        pltpu.sync_copy(rat_hbm.at[pl.ds(base, BPW)], rat_v)
        pltpu.sync_copy(wb_hbm, wb_v)

        copies = []
        for c in range(N_CHUNKS):
            sl = pl.ds(c * IDX_CHUNK, IDX_CHUNK)
            copies.append(pltpu.async_copy(
                otab_hbm.at[oid_v.at[sl]], orow_v.at[sl], sem))
            copies.append(pltpu.async_copy(
                ptab_hbm.at[pid_v.at[sl]], prow_v.at[sl], sem))
        for cp in copies:
            cp.wait()

        wvecs = [wb_v[pl.ds(k * L, L)] for k in range(112 // L)]

        def wsc(i):
            return wvecs[i // L][i % L]

        wo = [[wsc(d * OUT_DIM + j) for j in range(OUT_DIM)]
              for d in range(EMBED_DIM)]
        wp = [[wsc((EMBED_DIM + d) * OUT_DIM + j) for j in range(OUT_DIM)]
              for d in range(EMBED_DIM)]
        wr = [wsc(2 * EMBED_DIM * OUT_DIM + j) for j in range(OUT_DIM)]
        bb = [wsc((2 * EMBED_DIM + 1) * OUT_DIM + j) for j in range(OUT_DIM)]

        iota = lax.iota(jnp.int32, L)

        def group(g, carry):
            r0 = g * L
            rows = r0 + iota
            rat = rat_v[pl.ds(r0, L)]
            acc = [rat * wr[j] + bb[j] for j in range(OUT_DIM)]
            for d in range(EMBED_DIM):
                dsplat = jnp.full((L,), d, dtype=jnp.int32)
                co = plsc.load_gather(orow_v, [rows, dsplat])
                cpp = plsc.load_gather(prow_v, [rows, dsplat])
                for j in range(OUT_DIM):
                    acc[j] = acc[j] + co * wo[d][j] + cpp * wp[d][j]
            for j in range(OUT_DIM):
                plsc.store_scatter(
                    out_v, [rows, jnp.full((L,), j, dtype=jnp.int32)], acc[j])
            return carry

        lax.fori_loop(0, BPW // L, group, 0)
        pltpu.sync_copy(out_v, out_hbm.at[pl.ds(base, BPW)])

    return sc_call


def kernel(opening_input, player_input, rating_input, player_table,
           opening_table, W, b):
    oid = opening_input.reshape(-1).astype(jnp.int32)
    pid = player_input.reshape(-1).astype(jnp.int32)
    wb = jnp.zeros((112,), jnp.float32)
    wb = wb.at[:(2 * EMBED_DIM + 1) * OUT_DIM].set(
        W.astype(jnp.float32).reshape(-1))
    wb = wb.at[(2 * EMBED_DIM + 1) * OUT_DIM:
               (2 * EMBED_DIM + 2) * OUT_DIM].set(b.astype(jnp.float32))
    return _build_sc_call()(oid, pid, rating_input.astype(jnp.float32),
                            player_table, opening_table, wb)
